# Initial kernel scaffold; baseline (speedup 1.0000x reference)
import functools

import jax
import jax.numpy as jnp
from jax import lax
from jax.experimental import pallas as pl
from jax.experimental.pallas import tpu as pltpu
from jax.experimental.pallas import tpu_sc as plsc

N_USERS = 10000
N_ITEMS = 40000
N = N_USERS + N_ITEMS
DIM = 64
HALF = 32
LAYERS = 3
E = 800000
BATCH = 4096

NC = 2
NS = 16
CH = 128
NCHUNK = E // CH
ITERS = (NCHUNK + NS - 1) // NS
RPT = N // NS
ZCH = 625
PPT = BATCH // NS
PCH = PPT // CH


def _sc_body(e0, colb, rowb, wb, usersb, itemsb,
             e1, e2, e3, scores,
             acc, cidx, ridx, wbuf, rows, zbuf, ub, ib, sbuf):
    c = lax.axis_index("c")
    s = lax.axis_index("s")

    def zinit(r, carry):
        zbuf[r, pl.ds(0, 16)] = jnp.zeros((16,), jnp.float32)
        zbuf[r, pl.ds(16, 16)] = jnp.zeros((16,), jnp.float32)
        return carry
    lax.fori_loop(0, ZCH, zinit, 0, unroll=4)
    for k in range(RPT // ZCH):
        pltpu.sync_copy(zbuf, acc.at[pl.ds(s * RPT + k * ZCH, ZCH), :])
    plsc.subcore_barrier()

    def layer(src, dst):
        def chunk(i, carry):
            j = s + NS * i

            @pl.when(j < NCHUNK)
            def _():
                pltpu.sync_copy(colb.at[c, j], cidx)
                pltpu.sync_copy(rowb.at[j], ridx)
                pltpu.sync_copy(wb.at[j], wbuf)
                pltpu.sync_copy(src.at[cidx], rows)

                def mul(e, carry2):
                    ws = wbuf[e]
                    rows[e, pl.ds(0, 16)] = rows[e, pl.ds(0, 16)] * ws
                    rows[e, pl.ds(16, 16)] = rows[e, pl.ds(16, 16)] * ws
                    return carry2
                lax.fori_loop(0, CH, mul, 0, unroll=8)
                pltpu.sync_copy(rows, acc.at[ridx], add=True)
            return carry
        lax.fori_loop(0, ITERS, chunk, 0)
        plsc.subcore_barrier()
        for k in range(RPT // ZCH):
            r0 = s * RPT + k * ZCH
            pltpu.sync_copy(acc.at[pl.ds(r0, ZCH), :],
                            dst.at[pl.ds(c * N + r0, ZCH), :])
            pltpu.sync_copy(zbuf, acc.at[pl.ds(r0, ZCH), :])
        plsc.subcore_barrier()

    layer(e0, e1)
    layer(e1, e2)
    layer(e2, e3)

    for h in range(PCH):
        jrow = s * PCH + h
        for side, (idxb, buf) in enumerate(((usersb, ub), (itemsb, ib))):
            pltpu.sync_copy(idxb.at[c, jrow], cidx)
            for a, arr in enumerate((e0, e1, e2, e3)):
                dstslc = buf.at[pl.ds(h * CH, CH), :]
                if a == 0:
                    pltpu.sync_copy(arr.at[cidx], dstslc)
                else:
                    pltpu.sync_copy(arr.at[cidx], dstslc, add=True)

    def dot(p, carry):
        t = (ub[p, pl.ds(0, 16)] * ib[p, pl.ds(0, 16)]
             + ub[p, pl.ds(16, 16)] * ib[p, pl.ds(16, 16)])
        sbuf[p] = jnp.sum(t) * jnp.float32(1.0 / 16.0)
        return carry
    lax.fori_loop(0, PPT, dot, 0, unroll=4)
    pltpu.sync_copy(sbuf, scores.at[c, pl.ds(s * PPT, PPT)])


_sc_call = functools.partial(
    pl.kernel,
    out_type=[
        jax.ShapeDtypeStruct((NC * N, HALF), jnp.float32),
        jax.ShapeDtypeStruct((NC * N, HALF), jnp.float32),
        jax.ShapeDtypeStruct((NC * N, HALF), jnp.float32),
        jax.ShapeDtypeStruct((NC, BATCH), jnp.float32),
    ],
    mesh=plsc.VectorSubcoreMesh(core_axis_name="c", subcore_axis_name="s"),
    scratch_types=[
        pltpu.VMEM_SHARED((N, HALF), jnp.float32),
        pltpu.VMEM((CH,), jnp.int32),
        pltpu.VMEM((CH,), jnp.int32),
        pltpu.VMEM((CH,), jnp.float32),
        pltpu.VMEM((CH, HALF), jnp.float32),
        pltpu.VMEM((ZCH, HALF), jnp.float32),
        pltpu.VMEM((PPT, HALF), jnp.float32),
        pltpu.VMEM((PPT, HALF), jnp.float32),
        pltpu.VMEM((PPT,), jnp.float32),
    ],
)(_sc_body)


def kernel(users, items, user_emb, item_emb, edge_index, edge_weight):
    row = edge_index[0]
    col = edge_index[1]
    all_emb = jnp.concatenate([user_emb, item_emb], axis=0)
    e0 = all_emb.reshape(N, NC, HALF).transpose(1, 0, 2).reshape(NC * N, HALF)
    colb = jnp.stack([col, col + N]).reshape(NC, NCHUNK, CH)
    rowb = row.reshape(NCHUNK, CH)
    wb = edge_weight.reshape(NCHUNK, CH)
    usersb = jnp.stack([users, users + N]).reshape(NC, BATCH // CH, CH)
    itemsb = jnp.stack([items + N_USERS, items + N_USERS + N]).reshape(
        NC, BATCH // CH, CH)
    _, _, _, partial = _sc_call(e0, colb, rowb, wb, usersb, itemsb)
    return partial[0] + partial[1]


# speedup vs baseline: 3.5898x; 3.5898x over previous
import functools

import jax
import jax.numpy as jnp
from jax import lax
from jax.experimental import pallas as pl
from jax.experimental.pallas import tpu as pltpu
from jax.experimental.pallas import tpu_sc as plsc

N_USERS = 10000
N_ITEMS = 40000
N = N_USERS + N_ITEMS
DIM = 64
HALF = 32
LAYERS = 3
E = 800000
BATCH = 4096

NC = 2
NS = 16
CH = 128
NCHUNK = E // CH
ITERS = (NCHUNK + NS - 1) // NS
RPT = N // NS
ZCH = 125
PPT = BATCH // NS
PCH = PPT // CH


def _sc_body(e0, colb, rowb, wb, usersb, itemsb,
             e1, e2, e3, scores,
             acc, cidx, ridx, wbuf, rows, zbuf, ub, ib, sbuf):
    c = lax.axis_index("c")
    s = lax.axis_index("s")

    def zinit(r, carry):
        zbuf[r, pl.ds(0, 16)] = jnp.zeros((16,), jnp.float32)
        zbuf[r, pl.ds(16, 16)] = jnp.zeros((16,), jnp.float32)
        return carry
    lax.fori_loop(0, ZCH, zinit, 0, unroll=4)
    for k in range(RPT // ZCH):
        pltpu.sync_copy(zbuf, acc.at[pl.ds(s * RPT + k * ZCH, ZCH), :])
    plsc.subcore_barrier()

    def layer(src, dst):
        def chunk(i, carry):
            j = s + NS * i

            @pl.when(j < NCHUNK)
            def _():
                pltpu.sync_copy(colb.at[pl.ds(c * E + j * CH, CH)], cidx)
                pltpu.sync_copy(rowb.at[pl.ds(j * CH, CH)], ridx)
                pltpu.sync_copy(wb.at[pl.ds(j * CH, CH)], wbuf)
                pltpu.sync_copy(src.at[cidx], rows)

                def mul(g, carry2):
                    wv = wbuf[pl.ds(g * 16, 16)]
                    for t in range(16):
                        ws = wv[t]
                        e = g * 16 + t
                        rows[e, pl.ds(0, 16)] = rows[e, pl.ds(0, 16)] * ws
                        rows[e, pl.ds(16, 16)] = rows[e, pl.ds(16, 16)] * ws
                    return carry2
                lax.fori_loop(0, CH // 16, mul, 0)
                pltpu.sync_copy(rows, acc.at[ridx], add=True)
            return carry
        lax.fori_loop(0, ITERS, chunk, 0)
        plsc.subcore_barrier()
        for k in range(RPT // ZCH):
            r0 = s * RPT + k * ZCH
            pltpu.sync_copy(acc.at[pl.ds(r0, ZCH), :],
                            dst.at[pl.ds(c * N + r0, ZCH), :])
            pltpu.sync_copy(zbuf, acc.at[pl.ds(r0, ZCH), :])
        plsc.subcore_barrier()

    layer(e0, e1)
    layer(e1, e2)
    layer(e2, e3)

    for h in range(PCH):
        jrow = s * PCH + h
        for side, (idxb, buf) in enumerate(((usersb, ub), (itemsb, ib))):
            pltpu.sync_copy(idxb.at[pl.ds(c * BATCH + jrow * CH, CH)], cidx)
            for a, arr in enumerate((e0, e1, e2, e3)):
                dstslc = buf.at[pl.ds(h * CH, CH), :]
                if a == 0:
                    pltpu.sync_copy(arr.at[cidx], dstslc)
                else:
                    pltpu.sync_copy(arr.at[cidx], dstslc, add=True)

    lanes = lax.iota(jnp.int32, 16)

    def dot(g, carry):
        res = jnp.zeros((16,), jnp.float32)
        for t in range(16):
            p = g * 16 + t
            prod = (ub[p, pl.ds(0, 16)] * ib[p, pl.ds(0, 16)]
                    + ub[p, pl.ds(16, 16)] * ib[p, pl.ds(16, 16)])
            val = jnp.sum(prod) * jnp.float32(1.0 / 16.0)
            res = jnp.where(lanes == t, val, res)
        sbuf[pl.ds(g * 16, 16)] = res
        return carry
    lax.fori_loop(0, PPT // 16, dot, 0)
    pltpu.sync_copy(sbuf, scores.at[pl.ds(c * BATCH + s * PPT, PPT)])


_sc_call = functools.partial(
    pl.kernel,
    out_type=[
        jax.ShapeDtypeStruct((NC * N, HALF), jnp.float32),
        jax.ShapeDtypeStruct((NC * N, HALF), jnp.float32),
        jax.ShapeDtypeStruct((NC * N, HALF), jnp.float32),
        jax.ShapeDtypeStruct((NC * BATCH,), jnp.float32),
    ],
    mesh=plsc.VectorSubcoreMesh(core_axis_name="c", subcore_axis_name="s"),
    compiler_params=pltpu.CompilerParams(use_tc_tiling_on_sc=False,
                                         needs_layout_passes=False),
    scratch_types=[
        pltpu.VMEM_SHARED((N, HALF), jnp.float32),
        pltpu.VMEM((CH,), jnp.int32),
        pltpu.VMEM((CH,), jnp.int32),
        pltpu.VMEM((CH,), jnp.float32),
        pltpu.VMEM((CH, HALF), jnp.float32),
        pltpu.VMEM((ZCH, HALF), jnp.float32),
        pltpu.VMEM((PPT, HALF), jnp.float32),
        pltpu.VMEM((PPT, HALF), jnp.float32),
        pltpu.VMEM((PPT,), jnp.float32),
    ],
)(_sc_body)


def kernel(users, items, user_emb, item_emb, edge_index, edge_weight):
    row = edge_index[0]
    col = edge_index[1]
    all_emb = jnp.concatenate([user_emb, item_emb], axis=0)
    e0 = all_emb.reshape(N, NC, HALF).transpose(1, 0, 2).reshape(NC * N, HALF)
    colb = jnp.concatenate([col, col + N])
    usersb = jnp.concatenate([users, users + N])
    itemsb = jnp.concatenate([items + N_USERS, items + N_USERS + N])
    _, _, _, partial = _sc_call(e0, colb, row, edge_weight, usersb, itemsb)
    return partial[:BATCH] + partial[BATCH:]


# speedup vs baseline: 5.7167x; 1.5925x over previous
import functools

import jax
import jax.numpy as jnp
from jax import lax
from jax.experimental import pallas as pl
from jax.experimental.pallas import tpu as pltpu
from jax.experimental.pallas import tpu_sc as plsc

N_USERS = 10000
N_ITEMS = 40000
N = N_USERS + N_ITEMS
DIM = 64
HALF = 32
LAYERS = 3
E = 800000
BATCH = 4096

NC = 2
NS = 16
CH = 128
BLK = 2 * CH
NBLK = E // BLK
OUTER = 98
RPT = N // NS
ZCH = 125
PPT = BATCH // NS


def _sc_body(e0, edata, usersb, itemsb,
             e1, e2, e3, scores,
             acc, ebuf, rowsb, zbuf, ub, ib2, sbuf, fbu, fbi, sem0, sem1):
    c = lax.axis_index("c")
    s = lax.axis_index("s")
    sems = (sem0, sem1)

    def zinit(r, carry):
        zbuf[r, pl.ds(0, 16)] = jnp.zeros((16,), jnp.float32)
        zbuf[r, pl.ds(16, 16)] = jnp.zeros((16,), jnp.float32)
        return carry
    lax.fori_loop(0, ZCH, zinit, 0, unroll=4)
    for k in range(RPT // ZCH):
        pltpu.sync_copy(zbuf, acc.at[pl.ds(s * RPT + k * ZCH, ZCH), :])
    plsc.subcore_barrier()

    def layer(src, dst):
        def stage(ib, b):
            jb = s + NS * ib

            @pl.when(jb < NBLK)
            def _():
                pltpu.sync_copy(edata.at[c * NBLK + jb], ebuf.at[b])
                for u in range(2):
                    pltpu.async_copy(
                        src.at[ebuf.at[b, u]],
                        rowsb.at[b, pl.ds(u * CH, CH), :], sems[b])

        def process(ib, b):
            jb = s + NS * ib

            @pl.when(jb < NBLK)
            def _():
                for u in range(2):
                    pltpu.make_async_copy(
                        src.at[ebuf.at[b, u]],
                        rowsb.at[b, pl.ds(u * CH, CH), :], sems[b]).wait()

                def mul(g, carry2):
                    for u in range(2):
                        wv = plsc.bitcast(ebuf[b, 4 + u, pl.ds(g * 16, 16)],
                                          jnp.float32)
                        for t in range(16):
                            ws = wv[t]
                            e = u * CH + g * 16 + t
                            rowsb[b, e, pl.ds(0, 16)] = \
                                rowsb[b, e, pl.ds(0, 16)] * ws
                            rowsb[b, e, pl.ds(16, 16)] = \
                                rowsb[b, e, pl.ds(16, 16)] * ws
                    return carry2
                lax.fori_loop(0, CH // 16, mul, 0)
                for u in range(2):
                    pltpu.sync_copy(rowsb.at[b, pl.ds(u * CH, CH), :],
                                    acc.at[ebuf.at[b, 2 + u]], add=True)

        stage(0, 0)

        def outer(t, carry):
            stage(2 * t + 1, 1)
            process(2 * t, 0)
            stage(2 * t + 2, 0)
            process(2 * t + 1, 1)
            return carry
        lax.fori_loop(0, OUTER, outer, 0)
        plsc.subcore_barrier()
        for k in range(RPT // ZCH):
            r0 = s * RPT + k * ZCH
            pltpu.sync_copy(acc.at[pl.ds(r0, ZCH), :],
                            dst.at[pl.ds(c * N + r0, ZCH), :])
            pltpu.sync_copy(zbuf, acc.at[pl.ds(r0, ZCH), :])
        plsc.subcore_barrier()

    layer(e0, e1)
    layer(e1, e2)
    layer(e2, e3)

    lanes = lax.iota(jnp.int32, 16)
    for h in range(2):
        base = c * BATCH + (s * 2 + h) * CH
        pltpu.sync_copy(usersb.at[pl.ds(base, CH)], fbu)
        pltpu.sync_copy(itemsb.at[pl.ds(base, CH)], fbi)
        for a, arr in enumerate((e0, e1, e2, e3)):
            pltpu.sync_copy(arr.at[fbu], ub, add=(a > 0))
            pltpu.sync_copy(arr.at[fbi], ib2, add=(a > 0))

        def dot(g, carry):
            res = jnp.zeros((16,), jnp.float32)
            for t in range(16):
                p = g * 16 + t
                prod = (ub[p, pl.ds(0, 16)] * ib2[p, pl.ds(0, 16)]
                        + ub[p, pl.ds(16, 16)] * ib2[p, pl.ds(16, 16)])
                val = jnp.sum(prod) * jnp.float32(1.0 / 16.0)
                res = jnp.where(lanes == t, val, res)
            sbuf[pl.ds(h * CH + g * 16, 16)] = res
            return carry
        lax.fori_loop(0, CH // 16, dot, 0)
    pltpu.sync_copy(sbuf, scores.at[pl.ds(c * BATCH + s * PPT, PPT)])


_sc_call = functools.partial(
    pl.kernel,
    out_type=[
        jax.ShapeDtypeStruct((NC * N, HALF), jnp.float32),
        jax.ShapeDtypeStruct((NC * N, HALF), jnp.float32),
        jax.ShapeDtypeStruct((NC * N, HALF), jnp.float32),
        jax.ShapeDtypeStruct((NC * BATCH,), jnp.float32),
    ],
    mesh=plsc.VectorSubcoreMesh(core_axis_name="c", subcore_axis_name="s"),
    compiler_params=pltpu.CompilerParams(use_tc_tiling_on_sc=False,
                                         needs_layout_passes=False),
    scratch_types=[
        pltpu.VMEM_SHARED((N, HALF), jnp.float32),
        pltpu.VMEM((2, 6, CH), jnp.int32),
        pltpu.VMEM((2, BLK, HALF), jnp.float32),
        pltpu.VMEM((ZCH, HALF), jnp.float32),
        pltpu.VMEM((CH, HALF), jnp.float32),
        pltpu.VMEM((CH, HALF), jnp.float32),
        pltpu.VMEM((PPT,), jnp.float32),
        pltpu.VMEM((CH,), jnp.int32),
        pltpu.VMEM((CH,), jnp.int32),
        pltpu.SemaphoreType.DMA,
        pltpu.SemaphoreType.DMA,
    ],
)(_sc_body)


def kernel(users, items, user_emb, item_emb, edge_index, edge_weight):
    row = edge_index[0]
    col = edge_index[1]
    all_emb = jnp.concatenate([user_emb, item_emb], axis=0)
    e0 = all_emb.reshape(N, NC, HALF).transpose(1, 0, 2).reshape(NC * N, HALF)
    rowp = row.reshape(NBLK, 2, CH)
    wp = lax.bitcast_convert_type(edge_weight, jnp.int32).reshape(NBLK, 2, CH)
    cores = []
    for c in range(NC):
        colp = (col + c * N).reshape(NBLK, 2, CH)
        cores.append(jnp.concatenate([colp, rowp, wp], axis=1))
    edata = jnp.concatenate(cores, axis=0)
    usersb = jnp.concatenate([users, users + N])
    itemsb = jnp.concatenate([items + N_USERS, items + N_USERS + N])
    _, _, _, partial = _sc_call(e0, edata, usersb, itemsb)
    return partial[:BATCH] + partial[BATCH:]


# speedup vs baseline: 5.8310x; 1.0200x over previous
import functools

import jax
import jax.numpy as jnp
from jax import lax
from jax.experimental import pallas as pl
from jax.experimental.pallas import tpu as pltpu
from jax.experimental.pallas import tpu_sc as plsc

N_USERS = 10000
N_ITEMS = 40000
N = N_USERS + N_ITEMS
DIM = 64
HALF = 32
LAYERS = 3
E = 800000
BATCH = 4096

NC = 2
NS = 16
CH = 128
BLK = 2 * CH
NBLK = E // BLK
OUTER = 98
RPT = N // NS
ZCH = 125
PPT = BATCH // NS


def _sc_body(e0, edata, usersb, itemsb,
             e1, e2, e3, scores,
             acc, ebuf, rowsb, zbuf, ub, ib2, sbuf, fbu, fbi,
             sem0, sem1, ssem0, ssem1):
    c = lax.axis_index("c")
    s = lax.axis_index("s")
    sems = (sem0, sem1)
    ssems = (ssem0, ssem1)

    def zinit(r, carry):
        zbuf[r, pl.ds(0, 16)] = jnp.zeros((16,), jnp.float32)
        zbuf[r, pl.ds(16, 16)] = jnp.zeros((16,), jnp.float32)
        return carry
    lax.fori_loop(0, ZCH, zinit, 0, unroll=4)
    for k in range(RPT // ZCH):
        pltpu.sync_copy(zbuf, acc.at[pl.ds(s * RPT + k * ZCH, ZCH), :])
    plsc.subcore_barrier()

    def layer(src, dst):
        def stage(ib, b):
            jb = s + NS * ib
            jprev = jb - 2 * NS

            @pl.when(jnp.logical_and(jprev >= 0, jprev < NBLK))
            def _():
                for u in range(2):
                    pltpu.make_async_copy(
                        rowsb.at[b, pl.ds(u * CH, CH), :],
                        acc.at[ebuf.at[b, 2 + u]], ssems[b]).wait()

            @pl.when(jb < NBLK)
            def _():
                pltpu.sync_copy(edata.at[c * NBLK + jb], ebuf.at[b])
                for u in range(2):
                    pltpu.async_copy(
                        src.at[ebuf.at[b, u]],
                        rowsb.at[b, pl.ds(u * CH, CH), :], sems[b])

        def process(ib, b):
            jb = s + NS * ib

            @pl.when(jb < NBLK)
            def _():
                for u in range(2):
                    pltpu.make_async_copy(
                        src.at[ebuf.at[b, u]],
                        rowsb.at[b, pl.ds(u * CH, CH), :], sems[b]).wait()

                def mul(g, carry2):
                    for u in range(2):
                        wv = plsc.bitcast(ebuf[b, 4 + u, pl.ds(g * 16, 16)],
                                          jnp.float32)
                        for t in range(16):
                            ws = wv[t]
                            e = u * CH + g * 16 + t
                            rowsb[b, e, pl.ds(0, 16)] = \
                                rowsb[b, e, pl.ds(0, 16)] * ws
                            rowsb[b, e, pl.ds(16, 16)] = \
                                rowsb[b, e, pl.ds(16, 16)] * ws
                    return carry2
                lax.fori_loop(0, CH // 16, mul, 0)
                for u in range(2):
                    pltpu.async_copy(rowsb.at[b, pl.ds(u * CH, CH), :],
                                     acc.at[ebuf.at[b, 2 + u]], ssems[b],
                                     add=True)

        stage(0, 0)

        def outer(t, carry):
            stage(2 * t + 1, 1)
            process(2 * t, 0)
            stage(2 * t + 2, 0)
            process(2 * t + 1, 1)
            return carry
        lax.fori_loop(0, OUTER, outer, 0)

        @pl.when(s + NS * 195 < NBLK)
        def _():
            for u in range(2):
                pltpu.make_async_copy(
                    rowsb.at[1, pl.ds(u * CH, CH), :],
                    acc.at[ebuf.at[1, 2 + u]], ssems[1]).wait()
        plsc.subcore_barrier()
        for k in range(RPT // ZCH):
            r0 = s * RPT + k * ZCH
            pltpu.sync_copy(acc.at[pl.ds(r0, ZCH), :],
                            dst.at[pl.ds(c * N + r0, ZCH), :])
            pltpu.sync_copy(zbuf, acc.at[pl.ds(r0, ZCH), :])
        plsc.subcore_barrier()

    layer(e0, e1)
    layer(e1, e2)
    layer(e2, e3)

    lanes = lax.iota(jnp.int32, 16)
    for h in range(2):
        base = c * BATCH + (s * 2 + h) * CH
        pltpu.sync_copy(usersb.at[pl.ds(base, CH)], fbu)
        pltpu.sync_copy(itemsb.at[pl.ds(base, CH)], fbi)
        for a, arr in enumerate((e0, e1, e2, e3)):
            pltpu.sync_copy(arr.at[fbu], ub, add=(a > 0))
            pltpu.sync_copy(arr.at[fbi], ib2, add=(a > 0))

        def dot(g, carry):
            res = jnp.zeros((16,), jnp.float32)
            for t in range(16):
                p = g * 16 + t
                prod = (ub[p, pl.ds(0, 16)] * ib2[p, pl.ds(0, 16)]
                        + ub[p, pl.ds(16, 16)] * ib2[p, pl.ds(16, 16)])
                val = jnp.sum(prod) * jnp.float32(1.0 / 16.0)
                res = jnp.where(lanes == t, val, res)
            sbuf[pl.ds(h * CH + g * 16, 16)] = res
            return carry
        lax.fori_loop(0, CH // 16, dot, 0)
    pltpu.sync_copy(sbuf, scores.at[pl.ds(c * BATCH + s * PPT, PPT)])


_sc_call = functools.partial(
    pl.kernel,
    out_type=[
        jax.ShapeDtypeStruct((NC * N, HALF), jnp.float32),
        jax.ShapeDtypeStruct((NC * N, HALF), jnp.float32),
        jax.ShapeDtypeStruct((NC * N, HALF), jnp.float32),
        jax.ShapeDtypeStruct((NC * BATCH,), jnp.float32),
    ],
    mesh=plsc.VectorSubcoreMesh(core_axis_name="c", subcore_axis_name="s"),
    compiler_params=pltpu.CompilerParams(use_tc_tiling_on_sc=False,
                                         needs_layout_passes=False),
    scratch_types=[
        pltpu.VMEM_SHARED((N, HALF), jnp.float32),
        pltpu.VMEM((2, 6, CH), jnp.int32),
        pltpu.VMEM((2, BLK, HALF), jnp.float32),
        pltpu.VMEM((ZCH, HALF), jnp.float32),
        pltpu.VMEM((CH, HALF), jnp.float32),
        pltpu.VMEM((CH, HALF), jnp.float32),
        pltpu.VMEM((PPT,), jnp.float32),
        pltpu.VMEM((CH,), jnp.int32),
        pltpu.VMEM((CH,), jnp.int32),
        pltpu.SemaphoreType.DMA,
        pltpu.SemaphoreType.DMA,
        pltpu.SemaphoreType.DMA,
        pltpu.SemaphoreType.DMA,
    ],
)(_sc_body)


def kernel(users, items, user_emb, item_emb, edge_index, edge_weight):
    row = edge_index[0]
    col = edge_index[1]
    all_emb = jnp.concatenate([user_emb, item_emb], axis=0)
    e0 = all_emb.reshape(N, NC, HALF).transpose(1, 0, 2).reshape(NC * N, HALF)
    rowp = row.reshape(NBLK, 2, CH)
    wp = lax.bitcast_convert_type(edge_weight, jnp.int32).reshape(NBLK, 2, CH)
    cores = []
    for c in range(NC):
        colp = (col + c * N).reshape(NBLK, 2, CH)
        cores.append(jnp.concatenate([colp, rowp, wp], axis=1))
    edata = jnp.concatenate(cores, axis=0)
    usersb = jnp.concatenate([users, users + N])
    itemsb = jnp.concatenate([items + N_USERS, items + N_USERS + N])
    _, _, _, partial = _sc_call(e0, edata, usersb, itemsb)
    return partial[:BATCH] + partial[BATCH:]


# speedup vs baseline: 14.3659x; 2.4637x over previous
import functools

import jax
import jax.numpy as jnp
from jax import lax
from jax.experimental import pallas as pl
from jax.experimental.pallas import tpu as pltpu
from jax.experimental.pallas import tpu_sc as plsc

N_USERS = 10000
N_ITEMS = 40000
N = N_USERS + N_ITEMS
DIM = 64
HALF = 32
LAYERS = 3
E = 800000
BATCH = 4096

NC = 2
NS = 16
CH = 128
BLK = 2 * CH
NBLK = E // BLK
OUTER = 98
RPT = N // NS
ZCH = 125
PPT = BATCH // NS


def _sc_body(e0, edata, usersb, itemsb,
             e1, e2, e3, scores,
             acc, ebuf, rowsb, zbuf, ub, ib2, sbuf, fbu, fbi,
             sem0, sem1, ssem0, ssem1):
    c = lax.axis_index("c")
    s = lax.axis_index("s")
    sems = (sem0, sem1)
    ssems = (ssem0, ssem1)

    def zinit(r, carry):
        zbuf[r, pl.ds(0, 16)] = jnp.zeros((16,), jnp.float32)
        zbuf[r, pl.ds(16, 16)] = jnp.zeros((16,), jnp.float32)
        return carry
    lax.fori_loop(0, ZCH, zinit, 0, unroll=4)
    for k in range(RPT // ZCH):
        pltpu.sync_copy(zbuf, acc.at[pl.ds(s * RPT + k * ZCH, ZCH), :])
    plsc.subcore_barrier()

    def layer(src, dst):
        def stage(ib, b):
            jb = s + NS * ib
            jprev = jb - 2 * NS

            @pl.when(jnp.logical_and(jprev >= 0, jprev < NBLK))
            def _():
                for u in range(2):
                    pltpu.make_async_copy(
                        rowsb.at[b, pl.ds(u * CH, CH), :],
                        acc.at[ebuf.at[b, 2 + u]], ssems[b]).wait()

            @pl.when(jb < NBLK)
            def _():
                pltpu.sync_copy(edata.at[c * NBLK + jb], ebuf.at[b])
                for u in range(2):
                    pltpu.async_copy(
                        src.at[ebuf.at[b, u]],
                        rowsb.at[b, pl.ds(u * CH, CH), :], sems[b])

        def process(ib, b):
            jb = s + NS * ib

            @pl.when(jb < NBLK)
            def _():
                for u in range(2):
                    pltpu.make_async_copy(
                        src.at[ebuf.at[b, u]],
                        rowsb.at[b, pl.ds(u * CH, CH), :], sems[b]).wait()

                @functools.partial(plsc.parallel_loop, 0, CH // 16, unroll=2)
                def mul(g):
                    for u in range(2):
                        wv = plsc.bitcast(ebuf[b, 4 + u, pl.ds(g * 16, 16)],
                                          jnp.float32)
                        for t in range(16):
                            ws = wv[t]
                            e = u * CH + g * 16 + t
                            rowsb[b, e, pl.ds(0, 16)] = \
                                rowsb[b, e, pl.ds(0, 16)] * ws
                            rowsb[b, e, pl.ds(16, 16)] = \
                                rowsb[b, e, pl.ds(16, 16)] * ws
                for u in range(2):
                    pltpu.async_copy(rowsb.at[b, pl.ds(u * CH, CH), :],
                                     acc.at[ebuf.at[b, 2 + u]], ssems[b],
                                     add=True)

        stage(0, 0)

        def outer(t, carry):
            stage(2 * t + 1, 1)
            process(2 * t, 0)
            stage(2 * t + 2, 0)
            process(2 * t + 1, 1)
            return carry
        lax.fori_loop(0, OUTER, outer, 0)

        @pl.when(s + NS * 195 < NBLK)
        def _():
            for u in range(2):
                pltpu.make_async_copy(
                    rowsb.at[1, pl.ds(u * CH, CH), :],
                    acc.at[ebuf.at[1, 2 + u]], ssems[1]).wait()
        plsc.subcore_barrier()
        for k in range(RPT // ZCH):
            r0 = s * RPT + k * ZCH
            pltpu.sync_copy(acc.at[pl.ds(r0, ZCH), :],
                            dst.at[pl.ds(c * N + r0, ZCH), :])
            pltpu.sync_copy(zbuf, acc.at[pl.ds(r0, ZCH), :])
        plsc.subcore_barrier()

    layer(e0, e1)
    layer(e1, e2)
    layer(e2, e3)

    lanes = lax.iota(jnp.int32, 16)
    for h in range(2):
        base = c * BATCH + (s * 2 + h) * CH
        pltpu.sync_copy(usersb.at[pl.ds(base, CH)], fbu)
        pltpu.sync_copy(itemsb.at[pl.ds(base, CH)], fbi)
        for a, arr in enumerate((e0, e1, e2, e3)):
            pltpu.sync_copy(arr.at[fbu], ub, add=(a > 0))
            pltpu.sync_copy(arr.at[fbi], ib2, add=(a > 0))

        def dot(g, carry):
            res = jnp.zeros((16,), jnp.float32)
            for t in range(16):
                p = g * 16 + t
                prod = (ub[p, pl.ds(0, 16)] * ib2[p, pl.ds(0, 16)]
                        + ub[p, pl.ds(16, 16)] * ib2[p, pl.ds(16, 16)])
                val = jnp.sum(prod) * jnp.float32(1.0 / 16.0)
                res = jnp.where(lanes == t, val, res)
            sbuf[pl.ds(h * CH + g * 16, 16)] = res
            return carry
        lax.fori_loop(0, CH // 16, dot, 0)
    pltpu.sync_copy(sbuf, scores.at[pl.ds(c * BATCH + s * PPT, PPT)])


_sc_call = functools.partial(
    pl.kernel,
    out_type=[
        jax.ShapeDtypeStruct((NC * N, HALF), jnp.float32),
        jax.ShapeDtypeStruct((NC * N, HALF), jnp.float32),
        jax.ShapeDtypeStruct((NC * N, HALF), jnp.float32),
        jax.ShapeDtypeStruct((NC * BATCH,), jnp.float32),
    ],
    mesh=plsc.VectorSubcoreMesh(core_axis_name="c", subcore_axis_name="s"),
    compiler_params=pltpu.CompilerParams(use_tc_tiling_on_sc=False,
                                         needs_layout_passes=False),
    scratch_types=[
        pltpu.VMEM_SHARED((N, HALF), jnp.float32),
        pltpu.VMEM((2, 6, CH), jnp.int32),
        pltpu.VMEM((2, BLK, HALF), jnp.float32),
        pltpu.VMEM((ZCH, HALF), jnp.float32),
        pltpu.VMEM((CH, HALF), jnp.float32),
        pltpu.VMEM((CH, HALF), jnp.float32),
        pltpu.VMEM((PPT,), jnp.float32),
        pltpu.VMEM((CH,), jnp.int32),
        pltpu.VMEM((CH,), jnp.int32),
        pltpu.SemaphoreType.DMA,
        pltpu.SemaphoreType.DMA,
        pltpu.SemaphoreType.DMA,
        pltpu.SemaphoreType.DMA,
    ],
)(_sc_body)


def kernel(users, items, user_emb, item_emb, edge_index, edge_weight):
    row = edge_index[0]
    col = edge_index[1]
    all_emb = jnp.concatenate([user_emb, item_emb], axis=0)
    e0 = all_emb.reshape(N, NC, HALF).transpose(1, 0, 2).reshape(NC * N, HALF)
    rowp = row.reshape(NBLK, 2, CH)
    wp = lax.bitcast_convert_type(edge_weight, jnp.int32).reshape(NBLK, 2, CH)
    cores = []
    for c in range(NC):
        colp = (col + c * N).reshape(NBLK, 2, CH)
        cores.append(jnp.concatenate([colp, rowp, wp], axis=1))
    edata = jnp.concatenate(cores, axis=0)
    usersb = jnp.concatenate([users, users + N])
    itemsb = jnp.concatenate([items + N_USERS, items + N_USERS + N])
    _, _, _, partial = _sc_call(e0, edata, usersb, itemsb)
    return partial[:BATCH] + partial[BATCH:]
